# baseline (speedup 1.0000x reference)
import functools

import jax
import jax.numpy as jnp
from jax import lax
from jax.experimental import pallas as pl
from jax.experimental.pallas import tpu as pltpu
from jax.experimental.pallas import tpu_sc as plsc

B, L, D = 1024, 50, 512
F = L + 2
_info = plsc.get_sparse_core_info()
_NC, _NS = _info.num_cores, _info.num_subcores
_NW = _NC * _NS
_BPW = B // _NW


@functools.partial(
    pl.kernel,
    out_type=(
        jax.ShapeDtypeStruct((B, D), jnp.float32),
        jax.ShapeDtypeStruct((B, D), jnp.float32),
    ),
    mesh=plsc.VectorSubcoreMesh(core_axis_name="c", subcore_axis_name="s"),
    scratch_types=[
        pltpu.VMEM((_BPW,), jnp.int32),
        pltpu.VMEM((_BPW,), jnp.int32),
        pltpu.VMEM((_BPW, D), jnp.float32),
        pltpu.VMEM((_BPW, D), jnp.float32),
        pltpu.SemaphoreType.DMA,
        pltpu.SemaphoreType.DMA,
    ],
)
def _sc_gather(gidx_hbm, didx_hbm, wg_hbm, wd_hbm, outg_hbm, outd_hbm,
               gidx_v, didx_v, grows_v, drows_v, sem_g, sem_d):
    wid = lax.axis_index("s") * _NC + lax.axis_index("c")
    base = wid * _BPW
    pltpu.sync_copy(gidx_hbm.at[pl.ds(base, _BPW)], gidx_v)
    pltpu.sync_copy(didx_hbm.at[pl.ds(base, _BPW)], didx_v)
    cg = pltpu.async_copy(wg_hbm.at[gidx_v], grows_v, sem_g)
    cd = pltpu.async_copy(wd_hbm.at[didx_v], drows_v, sem_d)
    cg.wait()
    cd.wait()
    pltpu.sync_copy(grows_v, outg_hbm.at[pl.ds(base, _BPW)])
    pltpu.sync_copy(drows_v, outd_hbm.at[pl.ds(base, _BPW)])


_BT = 768
_BS = B - _BT
_SPW = _BS // _NW

_BB = 16
_NB = 4
_C = _BT // _BB


def _tc_body(f_hbm, g_ref, d_ref, o_hbm,
             fb0, fb1, ob0, ob1, is0, is1, os0, os1):
    fbs, obs, iss, oss = (fb0, fb1), (ob0, ob1), (is0, is1), (os0, os1)

    def in_copy(i):
        q, s = i % 2, (i // 2) % _NB
        return pltpu.make_async_copy(f_hbm.at[pl.ds(i * _BB, _BB)],
                                     fbs[q].at[s], iss[q].at[s])

    def out_copy(i):
        q, s = i % 2, (i // 2) % _NB
        return pltpu.make_async_copy(obs[q].at[s],
                                     o_hbm.at[pl.ds(i * _BB, _BB)],
                                     oss[q].at[s])

    for i in range(2 * _NB):
        in_copy(i).start()
    for i in range(_C):
        in_copy(i).wait()
        if i >= 2 * _NB:
            out_copy(i - 2 * _NB).wait()
        q, s = i % 2, (i // 2) % _NB
        obs[q][s, :, 2:, :] = fbs[q][s]
        obs[q][s, :, 0, :] = g_ref[pl.ds(i * _BB, _BB), :]
        obs[q][s, :, 1, :] = d_ref[pl.ds(i * _BB, _BB), :]
        out_copy(i).start()
        if i + 2 * _NB < _C:
            in_copy(i + 2 * _NB).start()
    for i in range(_C - 2 * _NB, _C):
        out_copy(i).wait()


def _tc_assemble(feature, embg, embd):
    return pl.pallas_call(
        _tc_body,
        in_specs=[
            pl.BlockSpec(memory_space=pl.ANY),
            pl.BlockSpec((B, D), lambda: (0, 0)),
            pl.BlockSpec((B, D), lambda: (0, 0)),
        ],
        out_specs=pl.BlockSpec(memory_space=pl.ANY),
        out_shape=jax.ShapeDtypeStruct((_BT, F, D), jnp.float32),
        scratch_shapes=[
            pltpu.VMEM((_NB, _BB, L, D), jnp.float32),
            pltpu.VMEM((_NB, _BB, L, D), jnp.float32),
            pltpu.VMEM((_NB, _BB, F, D), jnp.float32),
            pltpu.VMEM((_NB, _BB, F, D), jnp.float32),
            pltpu.SemaphoreType.DMA((_NB,)),
            pltpu.SemaphoreType.DMA((_NB,)),
            pltpu.SemaphoreType.DMA((_NB,)),
            pltpu.SemaphoreType.DMA((_NB,)),
        ],
    )(feature, embg, embd)


@functools.partial(
    pl.kernel,
    out_type=jax.ShapeDtypeStruct((_BS, F, D), jnp.float32),
    mesh=plsc.VectorSubcoreMesh(core_axis_name="c", subcore_axis_name="s"),
    scratch_types=[
        pltpu.VMEM((_SPW, D), jnp.float32),
        pltpu.VMEM((_SPW, D), jnp.float32),
        pltpu.VMEM((2, L, D), jnp.float32),
        pltpu.VMEM((2, F, D), jnp.float32),
        pltpu.SemaphoreType.DMA,
        pltpu.SemaphoreType.DMA,
        pltpu.SemaphoreType.DMA((2,)),
        pltpu.SemaphoreType.DMA((2,)),
    ],
)
def _sc_copy(f_hbm, embg_hbm, embd_hbm, outb_hbm,
             ge, de, fbuf, obuf, sem_g, sem_d, sem_in, sem_out):
    wid = lax.axis_index("s") * _NC + lax.axis_index("c")
    ob = wid * _SPW
    cg = pltpu.async_copy(embg_hbm.at[pl.ds(_BT + ob, _SPW)], ge, sem_g)
    cd = pltpu.async_copy(embd_hbm.at[pl.ds(_BT + ob, _SPW)], de, sem_d)

    def in_copy(k):
        r = lax.rem(k, 2)
        return pltpu.make_async_copy(f_hbm.at[_BT + ob + k], fbuf.at[r],
                                     sem_in.at[r])

    def out_copy(k):
        r = lax.rem(k, 2)
        return pltpu.make_async_copy(obuf.at[r], outb_hbm.at[ob + k],
                                     sem_out.at[r])

    in_copy(0).start()
    in_copy(1).start()
    cg.wait()
    cd.wait()

    def step(k, _):
        r = lax.rem(k, 2)
        in_copy(k).wait()

        @pl.when(k >= 2)
        def _():
            out_copy(k - 2).wait()

        def shift_rows(j, _):
            j2 = 2 * j
            for i in range(D // 16):
                sl = pl.ds(i * 16, 16)
                obuf[r, j2 + 2, sl] = fbuf[r, j2, sl]
                obuf[r, j2 + 3, sl] = fbuf[r, j2 + 1, sl]
            return 0

        lax.fori_loop(0, L // 2, shift_rows, 0)
        for i in range(D // 16):
            sl = pl.ds(i * 16, 16)
            obuf[r, 0, sl] = ge[k, sl]
            obuf[r, 1, sl] = de[k, sl]
        out_copy(k).start()

        @pl.when(k + 2 < _SPW)
        def _():
            in_copy(k + 2).start()

        return 0

    lax.fori_loop(0, _SPW, step, 0)
    out_copy(_SPW - 2).wait()
    out_copy(_SPW - 1).wait()


def kernel(feature, genre_index, difficulty_index, W_genre, W_difficulty):
    gidx = genre_index.reshape(B).astype(jnp.int32)
    didx = difficulty_index.reshape(B).astype(jnp.int32)
    embg, embd = _sc_gather(gidx, didx, W_genre, W_difficulty)
    out_a = _tc_assemble(feature, embg, embd)
    out_b = _sc_copy(feature, embg, embd)
    return jnp.concatenate([out_a, out_b], axis=0)


# speedup vs baseline: 1.1970x; 1.1970x over previous
import functools

import jax
import jax.numpy as jnp
from jax import lax
from jax.experimental import pallas as pl
from jax.experimental.pallas import tpu as pltpu
from jax.experimental.pallas import tpu_sc as plsc

B, L, D = 1024, 50, 512
F = L + 2
_info = plsc.get_sparse_core_info()
_NC, _NS = _info.num_cores, _info.num_subcores
_NW = _NC * _NS
_BPW = B // _NW


@functools.partial(
    pl.kernel,
    out_type=(
        jax.ShapeDtypeStruct((B, D), jnp.float32),
        jax.ShapeDtypeStruct((B, D), jnp.float32),
    ),
    mesh=plsc.VectorSubcoreMesh(core_axis_name="c", subcore_axis_name="s"),
    scratch_types=[
        pltpu.VMEM((_BPW,), jnp.int32),
        pltpu.VMEM((_BPW,), jnp.int32),
        pltpu.VMEM((_BPW, D), jnp.float32),
        pltpu.VMEM((_BPW, D), jnp.float32),
        pltpu.SemaphoreType.DMA,
        pltpu.SemaphoreType.DMA,
    ],
)
def _sc_gather(gidx_hbm, didx_hbm, wg_hbm, wd_hbm, outg_hbm, outd_hbm,
               gidx_v, didx_v, grows_v, drows_v, sem_g, sem_d):
    wid = lax.axis_index("s") * _NC + lax.axis_index("c")
    base = wid * _BPW
    pltpu.sync_copy(gidx_hbm.at[pl.ds(base, _BPW)], gidx_v)
    pltpu.sync_copy(didx_hbm.at[pl.ds(base, _BPW)], didx_v)
    cg = pltpu.async_copy(wg_hbm.at[gidx_v], grows_v, sem_g)
    cd = pltpu.async_copy(wd_hbm.at[didx_v], drows_v, sem_d)
    cg.wait()
    cd.wait()
    pltpu.sync_copy(grows_v, outg_hbm.at[pl.ds(base, _BPW)])
    pltpu.sync_copy(drows_v, outd_hbm.at[pl.ds(base, _BPW)])


_BB = 16
_NB = 4
_C = B // _BB


def _tc_body(f_hbm, g_ref, d_ref, o_hbm,
             fb0, fb1, ob0, ob1, is0, is1, os0, os1):
    fbs, obs, iss, oss = (fb0, fb1), (ob0, ob1), (is0, is1), (os0, os1)

    def in_copy(i):
        q, s = i % 2, (i // 2) % _NB
        return pltpu.make_async_copy(f_hbm.at[pl.ds(i * _BB, _BB)],
                                     fbs[q].at[s], iss[q].at[s])

    def out_copy(i):
        q, s = i % 2, (i // 2) % _NB
        return pltpu.make_async_copy(obs[q].at[s],
                                     o_hbm.at[pl.ds(i * _BB, _BB)],
                                     oss[q].at[s])

    for i in range(2 * _NB):
        in_copy(i).start()
    for i in range(_C):
        in_copy(i).wait()
        if i >= 2 * _NB:
            out_copy(i - 2 * _NB).wait()
        q, s = i % 2, (i // 2) % _NB
        obs[q][s, :, 2:, :] = fbs[q][s]
        obs[q][s, :, 0, :] = g_ref[pl.ds(i * _BB, _BB), :]
        obs[q][s, :, 1, :] = d_ref[pl.ds(i * _BB, _BB), :]
        out_copy(i).start()
        if i + 2 * _NB < _C:
            in_copy(i + 2 * _NB).start()
    for i in range(_C - 2 * _NB, _C):
        out_copy(i).wait()


def _tc_assemble(feature, embg, embd):
    return pl.pallas_call(
        _tc_body,
        in_specs=[
            pl.BlockSpec(memory_space=pl.ANY),
            pl.BlockSpec((B, D), lambda: (0, 0)),
            pl.BlockSpec((B, D), lambda: (0, 0)),
        ],
        out_specs=pl.BlockSpec(memory_space=pl.ANY),
        out_shape=jax.ShapeDtypeStruct((B, F, D), jnp.float32),
        scratch_shapes=[
            pltpu.VMEM((_NB, _BB, L, D), jnp.float32),
            pltpu.VMEM((_NB, _BB, L, D), jnp.float32),
            pltpu.VMEM((_NB, _BB, F, D), jnp.float32),
            pltpu.VMEM((_NB, _BB, F, D), jnp.float32),
            pltpu.SemaphoreType.DMA((_NB,)),
            pltpu.SemaphoreType.DMA((_NB,)),
            pltpu.SemaphoreType.DMA((_NB,)),
            pltpu.SemaphoreType.DMA((_NB,)),
        ],
    )(feature, embg, embd)


def kernel(feature, genre_index, difficulty_index, W_genre, W_difficulty):
    gidx = genre_index.reshape(B).astype(jnp.int32)
    didx = difficulty_index.reshape(B).astype(jnp.int32)
    embg, embd = _sc_gather(gidx, didx, W_genre, W_difficulty)
    return _tc_assemble(feature, embg, embd)
